# baseline (speedup 1.0000x reference)
import functools

import jax
import jax.numpy as jnp
from jax import lax
from jax.experimental import pallas as pl
from jax.experimental.pallas import tpu as pltpu
from jax.experimental.pallas import tpu_sc as plsc

N = 10000
E = 160000
F = 256
NPAD = 10240
NPW = 320
NW = 32
BLK = 2000
NCH = BLK // 16
NBLK = E // BLK
NEPW = 5024
EPAD = NW * NEPW
NCHUNK = NEPW // 16


def _proj1_kernel(x_ref, w_ref, b_ref, d_ref, s_ref):
    p = jnp.dot(x_ref[...], w_ref[...], preferred_element_type=jnp.float32)
    p = p + b_ref[...]
    d_ref[...] = p[:, :512].astype(jnp.bfloat16)
    s_ref[...] = p[:, 512:].astype(jnp.bfloat16)


def _proj2_kernel(x_ref, agg_ref, w_ref, b_ref, h_ref, d_ref, s_ref):
    h = x_ref[...] + agg_ref[...].astype(jnp.float32)
    h = jnp.where(h > 0, h, jnp.exp(h) - 1.0)
    h_ref[...] = h
    p = jnp.dot(h, w_ref[...], preferred_element_type=jnp.float32)
    p = p + b_ref[...]
    d_ref[...] = p[:, :512].astype(jnp.bfloat16)
    s_ref[...] = p[:, 512:].astype(jnp.bfloat16)


def _msg_kernel(z_ref, m_ref):
    zf = z_ref[:, :256].astype(jnp.float32)
    zs = z_ref[:, 256:].astype(jnp.float32)
    sig = 1.0 / (1.0 + jnp.exp(-zf))
    t = jnp.exp(-jnp.abs(zs))
    sp = jnp.maximum(zs, 0.0) + jnp.log(1.0 + t)
    m_ref[...] = (sig * sp).astype(jnp.bfloat16)


def _head_kernel(h_ref, agg_ref, w1_ref, b1_ref, w2_ref, b2_ref, o_ref):
    h2 = h_ref[...] + agg_ref[...].astype(jnp.float32)
    h2 = jnp.where(h2 > 0, h2, jnp.exp(h2) - 1.0)
    t = jnp.dot(h2, w1_ref[...], preferred_element_type=jnp.float32)
    t = t + b1_ref[...]
    t = jnp.where(t > 0, t, jnp.exp(t) - 1.0)
    o = jnp.dot(t, w2_ref[...], preferred_element_type=jnp.float32)
    o = o + b2_ref[...]
    lane = lax.broadcasted_iota(jnp.int32, o.shape, 1)
    valid = lane < 10
    neg = jnp.float32(-1e30)
    m = jnp.max(jnp.where(valid, o, neg), axis=1, keepdims=True)
    s = jnp.sum(jnp.where(valid, jnp.exp(o - m), 0.0), axis=1, keepdims=True)
    res = o - m - jnp.log(s)
    o_ref[...] = res[:, :10]


def _proj1(x, w, b):
    return pl.pallas_call(
        _proj1_kernel,
        grid=(NPAD // 256,),
        in_specs=[pl.BlockSpec((256, 256), lambda i: (i, 0)),
                  pl.BlockSpec((256, 1024), lambda i: (0, 0)),
                  pl.BlockSpec((1, 1024), lambda i: (0, 0))],
        out_specs=[pl.BlockSpec((256, 512), lambda i: (i, 0)),
                   pl.BlockSpec((256, 512), lambda i: (i, 0))],
        out_shape=[jax.ShapeDtypeStruct((NPAD, 512), jnp.bfloat16)] * 2,
    )(x, w, b)


def _proj2(x, agg, w, b):
    return pl.pallas_call(
        _proj2_kernel,
        grid=(NPAD // 256,),
        in_specs=[pl.BlockSpec((256, 256), lambda i: (i, 0)),
                  pl.BlockSpec((256, 256), lambda i: (i, 0)),
                  pl.BlockSpec((256, 1024), lambda i: (0, 0)),
                  pl.BlockSpec((1, 1024), lambda i: (0, 0))],
        out_specs=[pl.BlockSpec((256, 256), lambda i: (i, 0)),
                   pl.BlockSpec((256, 512), lambda i: (i, 0)),
                   pl.BlockSpec((256, 512), lambda i: (i, 0))],
        out_shape=[jax.ShapeDtypeStruct((N, 256), jnp.float32),
                   jax.ShapeDtypeStruct((NPAD, 512), jnp.bfloat16),
                   jax.ShapeDtypeStruct((NPAD, 512), jnp.bfloat16)],
    )(x, agg, w, b)


def _msg(z):
    return pl.pallas_call(
        _msg_kernel,
        grid=(EPAD // 512,),
        in_specs=[pl.BlockSpec((512, 512), lambda i: (i, 0))],
        out_specs=pl.BlockSpec((512, 256), lambda i: (i, 0)),
        out_shape=jax.ShapeDtypeStruct((EPAD, 256), jnp.bfloat16),
    )(z)


def _head(h, agg, w1, b1, w2, b2):
    return pl.pallas_call(
        _head_kernel,
        grid=(NPAD // 256,),
        in_specs=[pl.BlockSpec((256, 256), lambda i: (i, 0)),
                  pl.BlockSpec((256, 256), lambda i: (i, 0)),
                  pl.BlockSpec((256, 128), lambda i: (0, 0)),
                  pl.BlockSpec((1, 128), lambda i: (0, 0)),
                  pl.BlockSpec((128, 128), lambda i: (0, 0)),
                  pl.BlockSpec((1, 128), lambda i: (0, 0))],
        out_specs=pl.BlockSpec((256, 10), lambda i: (i, 0)),
        out_shape=jax.ShapeDtypeStruct((N, 10), jnp.float32),
    )(h, agg, w1, b1, w2, b2)


def _expand_body(d_hbm, s_hbm, dst_hbm, src_hbm, ea_hbm, wrow_hbm, z_hbm,
                 idd, ids, eas, dbuf, sbuf, zbuf, wrowv, sem0, sem1,
                 zsem0, zsem1):
    wid = lax.axis_index("s") * 2 + lax.axis_index("c")
    e0 = wid * NEPW

    pltpu.sync_copy(dst_hbm.at[pl.ds(e0, NEPW)], idd)
    pltpu.sync_copy(src_hbm.at[pl.ds(e0, NEPW)], ids)
    pltpu.sync_copy(ea_hbm.at[pl.ds(e0, NEPW)], eas)
    pltpu.sync_copy(wrow_hbm, wrowv)

    sems = (sem0, sem1)
    zsems = (zsem0, zsem1)

    def issue(c, slot):
        dv = idd[pl.ds(c * 16, 16)]
        sv = ids[pl.ds(c * 16, 16)]
        pltpu.async_copy(d_hbm.at[dv], dbuf.at[slot], sems[slot])
        pltpu.async_copy(s_hbm.at[sv], sbuf.at[slot], sems[slot])

    def wait(slot):
        pltpu.make_async_copy(d_hbm.at[pl.ds(0, 16)], dbuf.at[slot],
                              sems[slot]).wait()
        pltpu.make_async_copy(s_hbm.at[pl.ds(0, 16)], sbuf.at[slot],
                              sems[slot]).wait()

    def zdrain(slot):
        pltpu.make_async_copy(zbuf.at[slot], z_hbm.at[pl.ds(0, 16)],
                              zsems[slot]).wait()

    def compute(c, slot):
        @pl.when(c >= 2)
        def _():
            zdrain(slot)

        def edge_j(j, carry):
            eab = plsc.load_gather(eas, [jnp.full((16,), 0, jnp.int32)
                                         + (c * 16 + j)])
            eab2 = plsc.pack(eab, eab, format=plsc.PackFormat.INTERLEAVED)
            for v in range(16):
                sl = pl.ds(v * 16, 16)
                wsl = wrowv[0 if v < 8 else 1, pl.ds((v % 8) * 16, 16)]
                db = plsc.bitcast(dbuf[slot, j, sl], jnp.bfloat16)
                sb = plsc.bitcast(sbuf[slot, j, sl], jnp.bfloat16)
                wb = plsc.bitcast(wsl, jnp.bfloat16)
                z = db + sb + eab2 * wb
                zbuf[slot, j, sl] = plsc.bitcast(z, jnp.int32)
            return carry

        lax.fori_loop(0, 16, edge_j, 0)
        pltpu.async_copy(zbuf.at[slot], z_hbm.at[pl.ds(e0 + c * 16, 16)],
                         zsems[slot])

    issue(0, 0)

    def pair(p, carry):
        c0 = p * 2
        issue(c0 + 1, 1)
        wait(0)
        compute(c0, 0)

        @pl.when(c0 + 2 < NCHUNK)
        def _():
            issue(c0 + 2, 0)

        wait(1)
        compute(c0 + 1, 1)
        return carry

    lax.fori_loop(0, NCHUNK // 2, pair, 0)
    zdrain(0)
    zdrain(1)


_expand = functools.partial(
    pl.kernel,
    out_type=jax.ShapeDtypeStruct((EPAD, 256), jnp.int32),
    mesh=plsc.VectorSubcoreMesh(core_axis_name="c", subcore_axis_name="s"),
    compiler_params=pltpu.CompilerParams(needs_layout_passes=False),
    scratch_types=[
        pltpu.VMEM((NEPW,), jnp.int32),
        pltpu.VMEM((NEPW,), jnp.int32),
        pltpu.VMEM((NEPW,), jnp.float32),
        pltpu.VMEM((2, 16, 256), jnp.int32),
        pltpu.VMEM((2, 16, 256), jnp.int32),
        pltpu.VMEM((2, 16, 256), jnp.int32),
        pltpu.VMEM((2, 128), jnp.int32),
        pltpu.SemaphoreType.DMA,
        pltpu.SemaphoreType.DMA,
        pltpu.SemaphoreType.DMA,
        pltpu.SemaphoreType.DMA,
    ],
)(_expand_body)


def _segmax_body(msg_hbm, dst_hbm, out_hbm,
                 staging, blk, pend, dstv, mbuf, sem):
    wid = lax.axis_index("s") * 2 + lax.axis_index("c")
    base = wid * NPW

    zeros16 = jnp.zeros((16,), jnp.int32)

    def zrow(i, carry):
        for v in range(8):
            staging[i, pl.ds(v * 16, 16)] = zeros16
        return carry

    lax.fori_loop(0, NPW + 1, zrow, 0)

    sent = jnp.full((16,), E, jnp.int32)
    for q in range(3):
        pend[pl.ds(q * 16, 16)] = sent

    def rmw_group(g):
        def edge_j(j, carry):
            idxj = jnp.full((16,), 0, jnp.int32) + j
            dv = plsc.load_gather(dstv.at[g], [idxj])
            row = jnp.minimum(jnp.max(dv) - base, NPW)
            for v in range(8):
                sl = pl.ds(v * 16, 16)
                sb = plsc.bitcast(staging[row, sl], jnp.bfloat16)
                mb = plsc.bitcast(mbuf[g, j, sl], jnp.bfloat16)
                staging[row, sl] = plsc.bitcast(jnp.maximum(sb, mb),
                                                jnp.int32)
            return carry

        lax.fori_loop(0, 16, edge_j, 0)

    def process_two():
        e0 = pend[pl.ds(0, 16)]
        e1 = pend[pl.ds(16, 16)]
        c0 = pltpu.async_copy(dst_hbm.at[e0], dstv.at[0], sem)
        c1 = pltpu.async_copy(msg_hbm.at[e0], mbuf.at[0], sem)
        c2 = pltpu.async_copy(dst_hbm.at[e1], dstv.at[1], sem)
        c3 = pltpu.async_copy(msg_hbm.at[e1], mbuf.at[1], sem)
        c0.wait()
        c1.wait()
        rmw_group(0)
        c2.wait()
        c3.wait()
        rmw_group(1)

    def process_one(goff, g):
        eid = pend[pl.ds(goff, 16)]
        cp1 = pltpu.async_copy(dst_hbm.at[eid], dstv.at[g], sem)
        cp2 = pltpu.async_copy(msg_hbm.at[eid], mbuf.at[g], sem)
        cp1.wait()
        cp2.wait()
        rmw_group(g)

    def block_b(b, cnt):
        pltpu.sync_copy(dst_hbm.at[pl.ds(b * BLK, BLK)], blk)

        def chunk(c, cnt):
            dch = blk[pl.ds(c * 16, 16)]
            mask = (dch >= base) & (dch < base + NPW)
            cs = jnp.cumsum(jnp.where(mask, 1, 0))
            npc = jnp.max(cs)
            pos = cnt + cs - 1
            eidv = (b * BLK + c * 16) + lax.iota(jnp.int32, 16)
            plsc.store_scatter(pend, [pos], eidv, mask=mask)
            cnt = cnt + npc
            pl.when(cnt >= 32)(process_two)

            def shift():
                pend[pl.ds(0, 16)] = pend[pl.ds(32, 16)]

            pl.when(cnt >= 32)(shift)
            return jnp.where(cnt >= 32, cnt - 32, cnt)

        return lax.fori_loop(0, NCH, chunk, cnt)

    cnt = lax.fori_loop(0, NBLK, block_b, jnp.int32(0))
    pl.when(cnt > 0)(lambda: process_one(0, 0))
    pl.when(cnt > 16)(lambda: process_one(16, 1))

    @pl.when(wid < NW - 1)
    def _():
        pltpu.sync_copy(staging.at[pl.ds(0, NPW)], out_hbm.at[pl.ds(base, NPW)])

    @pl.when(wid == NW - 1)
    def _():
        pltpu.sync_copy(staging.at[pl.ds(0, N - (NW - 1) * NPW)],
                        out_hbm.at[pl.ds((NW - 1) * NPW, N - (NW - 1) * NPW)])


_segmax = functools.partial(
    pl.kernel,
    out_type=jax.ShapeDtypeStruct((N, 128), jnp.int32),
    mesh=plsc.VectorSubcoreMesh(core_axis_name="c", subcore_axis_name="s"),
    compiler_params=pltpu.CompilerParams(needs_layout_passes=False),
    scratch_types=[
        pltpu.VMEM((NPW + 1, 128), jnp.int32),
        pltpu.VMEM((BLK,), jnp.int32),
        pltpu.VMEM((48,), jnp.int32),
        pltpu.VMEM((2, 16), jnp.int32),
        pltpu.VMEM((2, 16, 128), jnp.int32),
        pltpu.SemaphoreType.DMA,
    ],
)(_segmax_body)


def _pack_i32(x):
    n, c = x.shape
    return lax.bitcast_convert_type(x.reshape(n, c // 2, 2), jnp.int32)


def _unpack_bf16(x):
    n, c = x.shape
    return lax.bitcast_convert_type(x, jnp.bfloat16).reshape(n, 2 * c)


def _layer_agg(d, s, dstp, srcp, eap, wrow):
    z = _expand(_pack_i32(d), _pack_i32(s), dstp, srcp, eap,
                _pack_i32(wrow.astype(jnp.bfloat16)))
    m = _msg(_unpack_bf16(z))
    agg = _segmax(_pack_i32(m), dstp)
    return _unpack_bf16(agg)


def kernel(x, edge_index, edge_attr, Wf1, bf1, Ws1, bs1, Wf2, bf2, Ws2, bs2,
           Wfc1, bfc1, Wfc2, bfc2):
    src = edge_index[0]
    dst = edge_index[1]
    npad = EPAD - E
    dstp = jnp.concatenate([dst, jnp.full((npad,), N, jnp.int32)])
    srcp = jnp.concatenate([src, jnp.zeros((npad,), jnp.int32)])
    eap = jnp.concatenate([edge_attr[:, 0], jnp.zeros((npad,), jnp.float32)])

    def wcat(Wf, bf, Ws, bs):
        w = jnp.concatenate([Wf[:F], Ws[:F], Wf[F:2 * F], Ws[F:2 * F]], axis=1)
        b = jnp.concatenate([bf, bs, jnp.zeros((512,), jnp.float32)])[None, :]
        wrow = jnp.stack([Wf[2 * F], Ws[2 * F]])
        return w, b, wrow

    w1, b1, wrow1 = wcat(Wf1, bf1, Ws1, bs1)
    w2, b2, wrow2 = wcat(Wf2, bf2, Ws2, bs2)

    d1, s1 = _proj1(x, w1, b1)
    agg1 = _layer_agg(d1, s1, dstp, srcp, eap, wrow1)
    h1, d2, s2 = _proj2(x, agg1, w2, b2)
    agg2 = _layer_agg(d2, s2, dstp, srcp, eap, wrow2)

    w2p = jnp.zeros((128, 128), jnp.float32).at[:, :10].set(Wfc2)
    b2p = jnp.zeros((1, 128), jnp.float32).at[0, :10].set(bfc2)
    return _head(h1, agg2, Wfc1, bfc1[None, :], w2p, b2p)


# speedup vs baseline: 2.4702x; 2.4702x over previous
import functools

import jax
import jax.numpy as jnp
from jax import lax
from jax.experimental import pallas as pl
from jax.experimental.pallas import tpu as pltpu
from jax.experimental.pallas import tpu_sc as plsc

N = 10000
E = 160000
F = 256
NPAD = 10240
NPW = 320
NW = 32
BLK = 2000
NCH = BLK // 16
NBLK = E // BLK
NEPW = 5024
EPAD = NW * NEPW
NCHUNK = NEPW // 16


def _pack_pair(lo, hi):
    lw = lax.bitcast_convert_type(lo.astype(jnp.bfloat16),
                                  jnp.uint16).astype(jnp.int32)
    hw = lax.bitcast_convert_type(hi.astype(jnp.bfloat16),
                                  jnp.uint16).astype(jnp.int32)
    return lw | (hw << 16)


def _unpack_lo(w):
    return lax.bitcast_convert_type(w << 16, jnp.float32)


def _unpack_hi(w):
    return lax.bitcast_convert_type(w & jnp.int32(-65536), jnp.float32)


def _proj1_kernel(x_ref, w_ref, b_ref, wr_ref, d_ref, s_ref, wrp_ref):
    p = jnp.dot(x_ref[...], w_ref[...], preferred_element_type=jnp.float32)
    p = p + b_ref[...]
    d_ref[...] = _pack_pair(p[:, :256], p[:, 256:512])
    s_ref[...] = _pack_pair(p[:, 512:768], p[:, 768:])
    wrp_ref[...] = _pack_pair(wr_ref[0:1, :], wr_ref[1:2, :])


def _proj2_kernel(x_ref, agg_ref, w_ref, b_ref, wr_ref, h_ref, d_ref,
                  s_ref, wrp_ref):
    aggw = agg_ref[...]
    agg = jnp.concatenate([_unpack_lo(aggw), _unpack_hi(aggw)], axis=1)
    h = x_ref[...] + agg
    h = jnp.where(h > 0, h, jnp.exp(h) - 1.0)
    h_ref[...] = h
    p = jnp.dot(h, w_ref[...], preferred_element_type=jnp.float32)
    p = p + b_ref[...]
    d_ref[...] = _pack_pair(p[:, :256], p[:, 256:512])
    s_ref[...] = _pack_pair(p[:, 512:768], p[:, 768:])
    wrp_ref[...] = _pack_pair(wr_ref[0:1, :], wr_ref[1:2, :])


def _msg_kernel(z_ref, m_ref):
    zw = z_ref[...]
    zf = _unpack_lo(zw)
    zs = _unpack_hi(zw)
    sig = 1.0 / (1.0 + jnp.exp(-zf))
    t = jnp.exp(-jnp.abs(zs))
    sp = jnp.maximum(zs, 0.0) + jnp.log(1.0 + t)
    m = sig * sp
    m_ref[...] = _pack_pair(m[:, :128], m[:, 128:])


def _head_kernel(h_ref, agg_ref, w1_ref, b1_ref, w2_ref, b2_ref, o_ref):
    aggw = agg_ref[...]
    agg = jnp.concatenate([_unpack_lo(aggw), _unpack_hi(aggw)], axis=1)
    h2 = h_ref[...] + agg
    h2 = jnp.where(h2 > 0, h2, jnp.exp(h2) - 1.0)
    t = jnp.dot(h2, w1_ref[...], preferred_element_type=jnp.float32)
    t = t + b1_ref[...]
    t = jnp.where(t > 0, t, jnp.exp(t) - 1.0)
    o = jnp.dot(t, w2_ref[...], preferred_element_type=jnp.float32)
    o = o + b2_ref[...]
    lane = lax.broadcasted_iota(jnp.int32, o.shape, 1)
    valid = lane < 10
    neg = jnp.float32(-1e30)
    m = jnp.max(jnp.where(valid, o, neg), axis=1, keepdims=True)
    s = jnp.sum(jnp.where(valid, jnp.exp(o - m), 0.0), axis=1, keepdims=True)
    res = o - m - jnp.log(s)
    o_ref[...] = res[:, :10]


def _proj1(x, w, b, wr):
    return pl.pallas_call(
        _proj1_kernel,
        grid=(NPAD // 256,),
        in_specs=[pl.BlockSpec((256, 256), lambda i: (i, 0)),
                  pl.BlockSpec((256, 1024), lambda i: (0, 0)),
                  pl.BlockSpec((1, 1024), lambda i: (0, 0)),
                  pl.BlockSpec((2, 256), lambda i: (0, 0))],
        out_specs=[pl.BlockSpec((256, 256), lambda i: (i, 0)),
                   pl.BlockSpec((256, 256), lambda i: (i, 0)),
                   pl.BlockSpec((1, 256), lambda i: (0, 0))],
        out_shape=[jax.ShapeDtypeStruct((NPAD, 256), jnp.int32),
                   jax.ShapeDtypeStruct((NPAD, 256), jnp.int32),
                   jax.ShapeDtypeStruct((1, 256), jnp.int32)],
    )(x, w, b, wr)


def _proj2(x, agg, w, b, wr):
    return pl.pallas_call(
        _proj2_kernel,
        grid=(NPAD // 256,),
        in_specs=[pl.BlockSpec((256, 256), lambda i: (i, 0)),
                  pl.BlockSpec((256, 128), lambda i: (i, 0)),
                  pl.BlockSpec((256, 1024), lambda i: (0, 0)),
                  pl.BlockSpec((1, 1024), lambda i: (0, 0)),
                  pl.BlockSpec((2, 256), lambda i: (0, 0))],
        out_specs=[pl.BlockSpec((256, 256), lambda i: (i, 0)),
                   pl.BlockSpec((256, 256), lambda i: (i, 0)),
                   pl.BlockSpec((256, 256), lambda i: (i, 0)),
                   pl.BlockSpec((1, 256), lambda i: (0, 0))],
        out_shape=[jax.ShapeDtypeStruct((N, 256), jnp.float32),
                   jax.ShapeDtypeStruct((NPAD, 256), jnp.int32),
                   jax.ShapeDtypeStruct((NPAD, 256), jnp.int32),
                   jax.ShapeDtypeStruct((1, 256), jnp.int32)],
    )(x, agg, w, b, wr)


def _msg(z):
    return pl.pallas_call(
        _msg_kernel,
        grid=(EPAD // 512,),
        in_specs=[pl.BlockSpec((512, 256), lambda i: (i, 0))],
        out_specs=pl.BlockSpec((512, 128), lambda i: (i, 0)),
        out_shape=jax.ShapeDtypeStruct((EPAD, 128), jnp.int32),
    )(z)


def _head(h, agg, w1, b1, w2, b2):
    return pl.pallas_call(
        _head_kernel,
        grid=(NPAD // 256,),
        in_specs=[pl.BlockSpec((256, 256), lambda i: (i, 0)),
                  pl.BlockSpec((256, 128), lambda i: (i, 0)),
                  pl.BlockSpec((256, 128), lambda i: (0, 0)),
                  pl.BlockSpec((1, 128), lambda i: (0, 0)),
                  pl.BlockSpec((128, 128), lambda i: (0, 0)),
                  pl.BlockSpec((1, 128), lambda i: (0, 0))],
        out_specs=pl.BlockSpec((256, 10), lambda i: (i, 0)),
        out_shape=jax.ShapeDtypeStruct((N, 10), jnp.float32),
    )(h, agg, w1, b1, w2, b2)


def _expand_body(d_hbm, s_hbm, dst_hbm, src_hbm, ea_hbm, wrow_hbm, z_hbm,
                 idd, ids, eas, dbuf, sbuf, zbuf, wrowv, sem0, sem1,
                 zsem0, zsem1):
    wid = lax.axis_index("s") * 2 + lax.axis_index("c")
    e0 = wid * NEPW

    pltpu.sync_copy(dst_hbm.at[pl.ds(e0, NEPW)], idd)
    pltpu.sync_copy(src_hbm.at[pl.ds(e0, NEPW)], ids)
    pltpu.sync_copy(ea_hbm.at[pl.ds(e0, NEPW)], eas)
    pltpu.sync_copy(wrow_hbm, wrowv)

    sems = (sem0, sem1)
    zsems = (zsem0, zsem1)

    def issue(c, slot):
        dv = idd[pl.ds(c * 16, 16)]
        sv = ids[pl.ds(c * 16, 16)]
        pltpu.async_copy(d_hbm.at[dv], dbuf.at[slot], sems[slot])
        pltpu.async_copy(s_hbm.at[sv], sbuf.at[slot], sems[slot])

    def wait(slot):
        pltpu.make_async_copy(d_hbm.at[pl.ds(0, 16)], dbuf.at[slot],
                              sems[slot]).wait()
        pltpu.make_async_copy(s_hbm.at[pl.ds(0, 16)], sbuf.at[slot],
                              sems[slot]).wait()

    def zdrain(slot):
        pltpu.make_async_copy(zbuf.at[slot], z_hbm.at[pl.ds(0, 16)],
                              zsems[slot]).wait()

    def compute(c, slot):
        @pl.when(c >= 2)
        def _():
            zdrain(slot)

        def edge_j(j, carry):
            eab = plsc.load_gather(eas, [jnp.full((16,), 0, jnp.int32)
                                         + (c * 16 + j)])
            eab2 = plsc.pack(eab, eab, format=plsc.PackFormat.INTERLEAVED)
            for v in range(16):
                sl = pl.ds(v * 16, 16)
                wsl = wrowv[0, pl.ds(v * 16, 16)]
                db = plsc.bitcast(dbuf[slot, j, sl], jnp.bfloat16)
                sb = plsc.bitcast(sbuf[slot, j, sl], jnp.bfloat16)
                wb = plsc.bitcast(wsl, jnp.bfloat16)
                z = db + sb + eab2 * wb
                zbuf[slot, j, sl] = plsc.bitcast(z, jnp.int32)
            return carry

        lax.fori_loop(0, 16, edge_j, 0)
        pltpu.async_copy(zbuf.at[slot], z_hbm.at[pl.ds(e0 + c * 16, 16)],
                         zsems[slot])

    issue(0, 0)

    def pair(p, carry):
        c0 = p * 2
        issue(c0 + 1, 1)
        wait(0)
        compute(c0, 0)

        @pl.when(c0 + 2 < NCHUNK)
        def _():
            issue(c0 + 2, 0)

        wait(1)
        compute(c0 + 1, 1)
        return carry

    lax.fori_loop(0, NCHUNK // 2, pair, 0)
    zdrain(0)
    zdrain(1)


_expand = functools.partial(
    pl.kernel,
    out_type=jax.ShapeDtypeStruct((EPAD, 256), jnp.int32),
    mesh=plsc.VectorSubcoreMesh(core_axis_name="c", subcore_axis_name="s"),
    compiler_params=pltpu.CompilerParams(needs_layout_passes=False),
    scratch_types=[
        pltpu.VMEM((NEPW,), jnp.int32),
        pltpu.VMEM((NEPW,), jnp.int32),
        pltpu.VMEM((NEPW,), jnp.float32),
        pltpu.VMEM((2, 16, 256), jnp.int32),
        pltpu.VMEM((2, 16, 256), jnp.int32),
        pltpu.VMEM((2, 16, 256), jnp.int32),
        pltpu.VMEM((1, 256), jnp.int32),
        pltpu.SemaphoreType.DMA,
        pltpu.SemaphoreType.DMA,
        pltpu.SemaphoreType.DMA,
        pltpu.SemaphoreType.DMA,
    ],
)(_expand_body)


def _segmax_body(msg_hbm, dst_hbm, out_hbm,
                 staging, blk, pend, dstv, mbuf, sem):
    wid = lax.axis_index("s") * 2 + lax.axis_index("c")
    base = wid * NPW

    zeros16 = jnp.zeros((16,), jnp.int32)

    def zrow(i, carry):
        for v in range(8):
            staging[i, pl.ds(v * 16, 16)] = zeros16
        return carry

    lax.fori_loop(0, NPW + 1, zrow, 0)

    sent = jnp.full((16,), E, jnp.int32)
    for q in range(3):
        pend[pl.ds(q * 16, 16)] = sent

    def rmw_group(g):
        def edge_j(j, carry):
            idxj = jnp.full((16,), 0, jnp.int32) + j
            dv = plsc.load_gather(dstv.at[g], [idxj])
            row = jnp.minimum(jnp.max(dv) - base, NPW)
            for v in range(8):
                sl = pl.ds(v * 16, 16)
                sb = plsc.bitcast(staging[row, sl], jnp.bfloat16)
                mb = plsc.bitcast(mbuf[g, j, sl], jnp.bfloat16)
                staging[row, sl] = plsc.bitcast(jnp.maximum(sb, mb),
                                                jnp.int32)
            return carry

        lax.fori_loop(0, 16, edge_j, 0)

    def process_two():
        e0 = pend[pl.ds(0, 16)]
        e1 = pend[pl.ds(16, 16)]
        c0 = pltpu.async_copy(dst_hbm.at[e0], dstv.at[0], sem)
        c1 = pltpu.async_copy(msg_hbm.at[e0], mbuf.at[0], sem)
        c2 = pltpu.async_copy(dst_hbm.at[e1], dstv.at[1], sem)
        c3 = pltpu.async_copy(msg_hbm.at[e1], mbuf.at[1], sem)
        c0.wait()
        c1.wait()
        rmw_group(0)
        c2.wait()
        c3.wait()
        rmw_group(1)

    def process_one(goff, g):
        eid = pend[pl.ds(goff, 16)]
        cp1 = pltpu.async_copy(dst_hbm.at[eid], dstv.at[g], sem)
        cp2 = pltpu.async_copy(msg_hbm.at[eid], mbuf.at[g], sem)
        cp1.wait()
        cp2.wait()
        rmw_group(g)

    def block_b(b, cnt):
        pltpu.sync_copy(dst_hbm.at[pl.ds(b * BLK, BLK)], blk)

        def chunk(c, cnt):
            dch = blk[pl.ds(c * 16, 16)]
            mask = (dch >= base) & (dch < base + NPW)
            cs = jnp.cumsum(jnp.where(mask, 1, 0))
            npc = jnp.max(cs)
            pos = cnt + cs - 1
            eidv = (b * BLK + c * 16) + lax.iota(jnp.int32, 16)
            plsc.store_scatter(pend, [pos], eidv, mask=mask)
            cnt = cnt + npc
            pl.when(cnt >= 32)(process_two)

            def shift():
                pend[pl.ds(0, 16)] = pend[pl.ds(32, 16)]

            pl.when(cnt >= 32)(shift)
            return jnp.where(cnt >= 32, cnt - 32, cnt)

        return lax.fori_loop(0, NCH, chunk, cnt)

    cnt = lax.fori_loop(0, NBLK, block_b, jnp.int32(0))
    pl.when(cnt > 0)(lambda: process_one(0, 0))
    pl.when(cnt > 16)(lambda: process_one(16, 1))

    @pl.when(wid < NW - 1)
    def _():
        pltpu.sync_copy(staging.at[pl.ds(0, NPW)], out_hbm.at[pl.ds(base, NPW)])

    @pl.when(wid == NW - 1)
    def _():
        pltpu.sync_copy(staging.at[pl.ds(0, N - (NW - 1) * NPW)],
                        out_hbm.at[pl.ds((NW - 1) * NPW, N - (NW - 1) * NPW)])


_segmax = functools.partial(
    pl.kernel,
    out_type=jax.ShapeDtypeStruct((N, 128), jnp.int32),
    mesh=plsc.VectorSubcoreMesh(core_axis_name="c", subcore_axis_name="s"),
    compiler_params=pltpu.CompilerParams(needs_layout_passes=False),
    scratch_types=[
        pltpu.VMEM((NPW + 1, 128), jnp.int32),
        pltpu.VMEM((BLK,), jnp.int32),
        pltpu.VMEM((48,), jnp.int32),
        pltpu.VMEM((2, 16), jnp.int32),
        pltpu.VMEM((2, 16, 128), jnp.int32),
        pltpu.SemaphoreType.DMA,
    ],
)(_segmax_body)


def _layer_agg(d, s, dstp, srcp, eap, wrowp):
    z = _expand(d, s, dstp, srcp, eap, wrowp)
    m = _msg(z)
    return _segmax(m, dstp)


def kernel(x, edge_index, edge_attr, Wf1, bf1, Ws1, bs1, Wf2, bf2, Ws2, bs2,
           Wfc1, bfc1, Wfc2, bfc2):
    src = edge_index[0]
    dst = edge_index[1]
    npad = EPAD - E
    dstp = jnp.concatenate([dst, jnp.full((npad,), N, jnp.int32)])
    srcp = jnp.concatenate([src, jnp.zeros((npad,), jnp.int32)])
    eap = jnp.concatenate([edge_attr[:, 0], jnp.zeros((npad,), jnp.float32)])

    def wcat(Wf, bf, Ws, bs):
        w = jnp.concatenate([Wf[:F], Ws[:F], Wf[F:2 * F], Ws[F:2 * F]], axis=1)
        b = jnp.concatenate([bf, bs, jnp.zeros((512,), jnp.float32)])[None, :]
        wrow = jnp.stack([Wf[2 * F], Ws[2 * F]])
        return w, b, wrow

    w1, b1, wrow1 = wcat(Wf1, bf1, Ws1, bs1)
    w2, b2, wrow2 = wcat(Wf2, bf2, Ws2, bs2)

    d1, s1, wrp1 = _proj1(x, w1, b1, wrow1)
    agg1 = _layer_agg(d1, s1, dstp, srcp, eap, wrp1)
    h1, d2, s2, wrp2 = _proj2(x, agg1, w2, b2, wrow2)
    agg2 = _layer_agg(d2, s2, dstp, srcp, eap, wrp2)

    w2p = jnp.zeros((128, 128), jnp.float32).at[:, :10].set(Wfc2)
    b2p = jnp.zeros((1, 128), jnp.float32).at[0, :10].set(bfc2)
    return _head(h1, agg2, Wfc1, bfc1[None, :], w2p, b2p)


# speedup vs baseline: 2.5956x; 1.0508x over previous
import functools

import jax
import jax.numpy as jnp
from jax import lax
from jax.experimental import pallas as pl
from jax.experimental.pallas import tpu as pltpu
from jax.experimental.pallas import tpu_sc as plsc

N = 10000
E = 160000
F = 256
NPAD = 10240
NPW = 320
NW = 32
BLK = 2000
NCH = BLK // 16
NBLK = E // BLK
NEPW = 5024
EPAD = NW * NEPW
NCHUNK = NEPW // 16


def _pack_pair(lo, hi):
    lw = lax.bitcast_convert_type(lo.astype(jnp.bfloat16),
                                  jnp.uint16).astype(jnp.int32)
    hw = lax.bitcast_convert_type(hi.astype(jnp.bfloat16),
                                  jnp.uint16).astype(jnp.int32)
    return lw | (hw << 16)


def _unpack_lo(w):
    return lax.bitcast_convert_type(w << 16, jnp.float32)


def _unpack_hi(w):
    return lax.bitcast_convert_type(w & jnp.int32(-65536), jnp.float32)


def _proj1_kernel(x_ref, w_ref, b_ref, wr_ref, d_ref, s_ref, wrp_ref):
    p = jnp.dot(x_ref[...], w_ref[...], preferred_element_type=jnp.float32)
    p = p + b_ref[...]
    d_ref[...] = _pack_pair(p[:, :256], p[:, 256:512])
    s_ref[...] = _pack_pair(p[:, 512:768], p[:, 768:])
    wrp_ref[...] = _pack_pair(wr_ref[0:1, :], wr_ref[1:2, :])


def _proj2_kernel(x_ref, agg_ref, w_ref, b_ref, wr_ref, h_ref, d_ref,
                  s_ref, wrp_ref):
    aggw = agg_ref[...]
    agg = jnp.concatenate([_unpack_lo(aggw), _unpack_hi(aggw)], axis=1)
    h = x_ref[...] + agg
    h = jnp.where(h > 0, h, jnp.exp(h) - 1.0)
    h_ref[...] = h
    p = jnp.dot(h, w_ref[...], preferred_element_type=jnp.float32)
    p = p + b_ref[...]
    d_ref[...] = _pack_pair(p[:, :256], p[:, 256:512])
    s_ref[...] = _pack_pair(p[:, 512:768], p[:, 768:])
    wrp_ref[...] = _pack_pair(wr_ref[0:1, :], wr_ref[1:2, :])


def _msg_kernel(z_ref, m_ref):
    zw = z_ref[...]
    zf = _unpack_lo(zw)
    zs = _unpack_hi(zw)
    sig = 1.0 / (1.0 + jnp.exp(-zf))
    t = jnp.exp(-jnp.abs(zs))
    sp = jnp.maximum(zs, 0.0) + jnp.log(1.0 + t)
    m = sig * sp
    m_ref[...] = _pack_pair(m[:, :128], m[:, 128:])


def _head_kernel(h_ref, agg_ref, w1_ref, b1_ref, w2_ref, b2_ref, o_ref):
    aggw = agg_ref[...]
    agg = jnp.concatenate([_unpack_lo(aggw), _unpack_hi(aggw)], axis=1)
    h2 = h_ref[...] + agg
    h2 = jnp.where(h2 > 0, h2, jnp.exp(h2) - 1.0)
    t = jnp.dot(h2, w1_ref[...], preferred_element_type=jnp.float32)
    t = t + b1_ref[...]
    t = jnp.where(t > 0, t, jnp.exp(t) - 1.0)
    o = jnp.dot(t, w2_ref[...], preferred_element_type=jnp.float32)
    o = o + b2_ref[...]
    lane = lax.broadcasted_iota(jnp.int32, o.shape, 1)
    valid = lane < 10
    neg = jnp.float32(-1e30)
    m = jnp.max(jnp.where(valid, o, neg), axis=1, keepdims=True)
    s = jnp.sum(jnp.where(valid, jnp.exp(o - m), 0.0), axis=1, keepdims=True)
    res = o - m - jnp.log(s)
    o_ref[...] = res[:, :10]


def _proj1(x, w, b, wr):
    return pl.pallas_call(
        _proj1_kernel,
        grid=(NPAD // 256,),
        in_specs=[pl.BlockSpec((256, 256), lambda i: (i, 0)),
                  pl.BlockSpec((256, 1024), lambda i: (0, 0)),
                  pl.BlockSpec((1, 1024), lambda i: (0, 0)),
                  pl.BlockSpec((2, 256), lambda i: (0, 0))],
        out_specs=[pl.BlockSpec((256, 256), lambda i: (i, 0)),
                   pl.BlockSpec((256, 256), lambda i: (i, 0)),
                   pl.BlockSpec((1, 256), lambda i: (0, 0))],
        out_shape=[jax.ShapeDtypeStruct((NPAD, 256), jnp.int32),
                   jax.ShapeDtypeStruct((NPAD, 256), jnp.int32),
                   jax.ShapeDtypeStruct((1, 256), jnp.int32)],
    )(x, w, b, wr)


def _proj2(x, agg, w, b, wr):
    return pl.pallas_call(
        _proj2_kernel,
        grid=(NPAD // 256,),
        in_specs=[pl.BlockSpec((256, 256), lambda i: (i, 0)),
                  pl.BlockSpec((256, 128), lambda i: (i, 0)),
                  pl.BlockSpec((256, 1024), lambda i: (0, 0)),
                  pl.BlockSpec((1, 1024), lambda i: (0, 0)),
                  pl.BlockSpec((2, 256), lambda i: (0, 0))],
        out_specs=[pl.BlockSpec((256, 256), lambda i: (i, 0)),
                   pl.BlockSpec((256, 256), lambda i: (i, 0)),
                   pl.BlockSpec((256, 256), lambda i: (i, 0)),
                   pl.BlockSpec((1, 256), lambda i: (0, 0))],
        out_shape=[jax.ShapeDtypeStruct((N, 256), jnp.float32),
                   jax.ShapeDtypeStruct((NPAD, 256), jnp.int32),
                   jax.ShapeDtypeStruct((NPAD, 256), jnp.int32),
                   jax.ShapeDtypeStruct((1, 256), jnp.int32)],
    )(x, agg, w, b, wr)


def _msg(z):
    return pl.pallas_call(
        _msg_kernel,
        grid=(EPAD // 512,),
        in_specs=[pl.BlockSpec((512, 256), lambda i: (i, 0))],
        out_specs=pl.BlockSpec((512, 128), lambda i: (i, 0)),
        out_shape=jax.ShapeDtypeStruct((EPAD, 128), jnp.int32),
    )(z)


def _head(h, agg, w1, b1, w2, b2):
    return pl.pallas_call(
        _head_kernel,
        grid=(NPAD // 256,),
        in_specs=[pl.BlockSpec((256, 256), lambda i: (i, 0)),
                  pl.BlockSpec((256, 128), lambda i: (i, 0)),
                  pl.BlockSpec((256, 128), lambda i: (0, 0)),
                  pl.BlockSpec((1, 128), lambda i: (0, 0)),
                  pl.BlockSpec((128, 128), lambda i: (0, 0)),
                  pl.BlockSpec((1, 128), lambda i: (0, 0))],
        out_specs=pl.BlockSpec((256, 10), lambda i: (i, 0)),
        out_shape=jax.ShapeDtypeStruct((N, 10), jnp.float32),
    )(h, agg, w1, b1, w2, b2)


def _expand_body(d_hbm, s_hbm, dst_hbm, src_hbm, ea_hbm, wrow_hbm, z_hbm,
                 idd, ids, eas, dbuf, sbuf, zbuf, wrowv, sem0, sem1,
                 zsem0, zsem1):
    wid = lax.axis_index("s") * 2 + lax.axis_index("c")
    e0 = wid * NEPW

    pltpu.sync_copy(dst_hbm.at[pl.ds(e0, NEPW)], idd)
    pltpu.sync_copy(src_hbm.at[pl.ds(e0, NEPW)], ids)
    pltpu.sync_copy(ea_hbm.at[pl.ds(e0, NEPW)], eas)
    pltpu.sync_copy(wrow_hbm, wrowv)

    sems = (sem0, sem1)
    zsems = (zsem0, zsem1)

    def issue(c, slot):
        dv = idd[pl.ds(c * 16, 16)]
        sv = ids[pl.ds(c * 16, 16)]
        pltpu.async_copy(d_hbm.at[dv], dbuf.at[slot], sems[slot])
        pltpu.async_copy(s_hbm.at[sv], sbuf.at[slot], sems[slot])

    def wait(slot):
        pltpu.make_async_copy(d_hbm.at[pl.ds(0, 16)], dbuf.at[slot],
                              sems[slot]).wait()
        pltpu.make_async_copy(s_hbm.at[pl.ds(0, 16)], sbuf.at[slot],
                              sems[slot]).wait()

    def zdrain(slot):
        pltpu.make_async_copy(zbuf.at[slot], z_hbm.at[pl.ds(0, 16)],
                              zsems[slot]).wait()

    def compute(c, slot):
        @pl.when(c >= 2)
        def _():
            zdrain(slot)

        def edge_j(j, carry):
            eab = plsc.load_gather(eas, [jnp.full((16,), 0, jnp.int32)
                                         + (c * 16 + j)])
            eab2 = plsc.pack(eab, eab, format=plsc.PackFormat.INTERLEAVED)
            for v in range(16):
                sl = pl.ds(v * 16, 16)
                wsl = wrowv[0, pl.ds(v * 16, 16)]
                db = plsc.bitcast(dbuf[slot, j, sl], jnp.bfloat16)
                sb = plsc.bitcast(sbuf[slot, j, sl], jnp.bfloat16)
                wb = plsc.bitcast(wsl, jnp.bfloat16)
                z = db + sb + eab2 * wb
                zbuf[slot, j, sl] = plsc.bitcast(z, jnp.int32)
            return carry

        lax.fori_loop(0, 16, edge_j, 0)
        pltpu.async_copy(zbuf.at[slot], z_hbm.at[pl.ds(e0 + c * 16, 16)],
                         zsems[slot])

    issue(0, 0)

    def pair(p, carry):
        c0 = p * 2
        issue(c0 + 1, 1)
        wait(0)
        compute(c0, 0)

        @pl.when(c0 + 2 < NCHUNK)
        def _():
            issue(c0 + 2, 0)

        wait(1)
        compute(c0 + 1, 1)
        return carry

    lax.fori_loop(0, NCHUNK // 2, pair, 0)
    zdrain(0)
    zdrain(1)


_expand = functools.partial(
    pl.kernel,
    out_type=jax.ShapeDtypeStruct((EPAD, 256), jnp.int32),
    mesh=plsc.VectorSubcoreMesh(core_axis_name="c", subcore_axis_name="s"),
    compiler_params=pltpu.CompilerParams(needs_layout_passes=False),
    scratch_types=[
        pltpu.VMEM((NEPW,), jnp.int32),
        pltpu.VMEM((NEPW,), jnp.int32),
        pltpu.VMEM((NEPW,), jnp.float32),
        pltpu.VMEM((2, 16, 256), jnp.int32),
        pltpu.VMEM((2, 16, 256), jnp.int32),
        pltpu.VMEM((2, 16, 256), jnp.int32),
        pltpu.VMEM((1, 256), jnp.int32),
        pltpu.SemaphoreType.DMA,
        pltpu.SemaphoreType.DMA,
        pltpu.SemaphoreType.DMA,
        pltpu.SemaphoreType.DMA,
    ],
)(_expand_body)


def _segmax_body(msg_hbm, dst_hbm, out_hbm,
                 staging, blk, pend, dstv, mbuf, sem):
    wid = lax.axis_index("s") * 2 + lax.axis_index("c")
    base = wid * NPW

    zeros16 = jnp.zeros((16,), jnp.int32)

    def zrow(i, carry):
        for v in range(8):
            staging[i, pl.ds(v * 16, 16)] = zeros16
        return carry

    lax.fori_loop(0, NPW + 1, zrow, 0)

    sent = jnp.full((16,), E, jnp.int32)
    for q in range(3):
        pend[pl.ds(q * 16, 16)] = sent

    def rmw_group(g):
        def edge_j(j, carry):
            idxj = jnp.full((16,), 0, jnp.int32) + j
            dv = plsc.load_gather(dstv.at[g], [idxj])
            d0 = lax.squeeze(lax.slice(dv, (0,), (1,)), (0,))
            row = jnp.minimum(d0 - base, NPW)
            for v in range(8):
                sl = pl.ds(v * 16, 16)
                sb = plsc.bitcast(staging[row, sl], jnp.bfloat16)
                mb = plsc.bitcast(mbuf[g, j, sl], jnp.bfloat16)
                staging[row, sl] = plsc.bitcast(jnp.maximum(sb, mb),
                                                jnp.int32)
            return carry

        lax.fori_loop(0, 16, edge_j, 0)

    def process_two():
        e0 = pend[pl.ds(0, 16)]
        e1 = pend[pl.ds(16, 16)]
        c0 = pltpu.async_copy(dst_hbm.at[e0], dstv.at[0], sem)
        c1 = pltpu.async_copy(msg_hbm.at[e0], mbuf.at[0], sem)
        c2 = pltpu.async_copy(dst_hbm.at[e1], dstv.at[1], sem)
        c3 = pltpu.async_copy(msg_hbm.at[e1], mbuf.at[1], sem)
        c0.wait()
        c1.wait()
        rmw_group(0)
        c2.wait()
        c3.wait()
        rmw_group(1)

    def process_one(goff, g):
        eid = pend[pl.ds(goff, 16)]
        cp1 = pltpu.async_copy(dst_hbm.at[eid], dstv.at[g], sem)
        cp2 = pltpu.async_copy(msg_hbm.at[eid], mbuf.at[g], sem)
        cp1.wait()
        cp2.wait()
        rmw_group(g)

    def block_b(b, cnt):
        pltpu.sync_copy(dst_hbm.at[pl.ds(b * BLK, BLK)], blk)

        def chunk(c, cnt):
            dch = blk[pl.ds(c * 16, 16)]
            mask = (dch >= base) & (dch < base + NPW)
            cs = jnp.cumsum(jnp.where(mask, 1, 0))
            npc = lax.squeeze(lax.slice(cs, (15,), (16,)), (0,))
            pos = cnt + cs - 1
            eidv = (b * BLK + c * 16) + lax.iota(jnp.int32, 16)
            plsc.store_scatter(pend, [pos], eidv, mask=mask)
            cnt = cnt + npc
            pl.when(cnt >= 32)(process_two)

            def shift():
                pend[pl.ds(0, 16)] = pend[pl.ds(32, 16)]

            pl.when(cnt >= 32)(shift)
            return jnp.where(cnt >= 32, cnt - 32, cnt)

        return lax.fori_loop(0, NCH, chunk, cnt)

    cnt = lax.fori_loop(0, NBLK, block_b, jnp.int32(0))
    pl.when(cnt > 0)(lambda: process_one(0, 0))
    pl.when(cnt > 16)(lambda: process_one(16, 1))

    @pl.when(wid < NW - 1)
    def _():
        pltpu.sync_copy(staging.at[pl.ds(0, NPW)], out_hbm.at[pl.ds(base, NPW)])

    @pl.when(wid == NW - 1)
    def _():
        pltpu.sync_copy(staging.at[pl.ds(0, N - (NW - 1) * NPW)],
                        out_hbm.at[pl.ds((NW - 1) * NPW, N - (NW - 1) * NPW)])


_segmax = functools.partial(
    pl.kernel,
    out_type=jax.ShapeDtypeStruct((N, 128), jnp.int32),
    mesh=plsc.VectorSubcoreMesh(core_axis_name="c", subcore_axis_name="s"),
    compiler_params=pltpu.CompilerParams(needs_layout_passes=False),
    scratch_types=[
        pltpu.VMEM((NPW + 1, 128), jnp.int32),
        pltpu.VMEM((BLK,), jnp.int32),
        pltpu.VMEM((48,), jnp.int32),
        pltpu.VMEM((2, 16), jnp.int32),
        pltpu.VMEM((2, 16, 128), jnp.int32),
        pltpu.SemaphoreType.DMA,
    ],
)(_segmax_body)


def _layer_agg(d, s, dstp, srcp, eap, wrowp):
    z = _expand(d, s, dstp, srcp, eap, wrowp)
    m = _msg(z)
    return _segmax(m, dstp)


def kernel(x, edge_index, edge_attr, Wf1, bf1, Ws1, bs1, Wf2, bf2, Ws2, bs2,
           Wfc1, bfc1, Wfc2, bfc2):
    src = edge_index[0]
    dst = edge_index[1]
    npad = EPAD - E
    dstp = jnp.concatenate([dst, jnp.full((npad,), N, jnp.int32)])
    srcp = jnp.concatenate([src, jnp.zeros((npad,), jnp.int32)])
    eap = jnp.concatenate([edge_attr[:, 0], jnp.zeros((npad,), jnp.float32)])

    def wcat(Wf, bf, Ws, bs):
        w = jnp.concatenate([Wf[:F], Ws[:F], Wf[F:2 * F], Ws[F:2 * F]], axis=1)
        b = jnp.concatenate([bf, bs, jnp.zeros((512,), jnp.float32)])[None, :]
        wrow = jnp.stack([Wf[2 * F], Ws[2 * F]])
        return w, b, wrow

    w1, b1, wrow1 = wcat(Wf1, bf1, Ws1, bs1)
    w2, b2, wrow2 = wcat(Wf2, bf2, Ws2, bs2)

    d1, s1, wrp1 = _proj1(x, w1, b1, wrow1)
    agg1 = _layer_agg(d1, s1, dstp, srcp, eap, wrp1)
    h1, d2, s2, wrp2 = _proj2(x, agg1, w2, b2, wrow2)
    agg2 = _layer_agg(d2, s2, dstp, srcp, eap, wrp2)

    w2p = jnp.zeros((128, 128), jnp.float32).at[:, :10].set(Wfc2)
    b2p = jnp.zeros((1, 128), jnp.float32).at[0, :10].set(bfc2)
    return _head(h1, agg2, Wfc1, bfc1[None, :], w2p, b2p)


# speedup vs baseline: 2.6088x; 1.0051x over previous
import functools

import jax
import jax.numpy as jnp
from jax import lax
from jax.experimental import pallas as pl
from jax.experimental.pallas import tpu as pltpu
from jax.experimental.pallas import tpu_sc as plsc

N = 10000
E = 160000
F = 256
NPAD = 10240
NPW = 320
NW = 32
BLK = 2000
NCH = BLK // 16
NBLK = E // BLK
NEPW = 5024
EPAD = NW * NEPW
NCHUNK = NEPW // 16


def _pack_pair(lo, hi):
    lw = lax.bitcast_convert_type(lo.astype(jnp.bfloat16),
                                  jnp.uint16).astype(jnp.int32)
    hw = lax.bitcast_convert_type(hi.astype(jnp.bfloat16),
                                  jnp.uint16).astype(jnp.int32)
    return lw | (hw << 16)


def _unpack_lo(w):
    return lax.bitcast_convert_type(w << 16, jnp.float32)


def _unpack_hi(w):
    return lax.bitcast_convert_type(w & jnp.int32(-65536), jnp.float32)


def _proj1_kernel(x_ref, w_ref, b_ref, wr_ref, d_ref, s_ref, wrp_ref):
    p = jnp.dot(x_ref[...], w_ref[...], preferred_element_type=jnp.float32)
    p = p + b_ref[...]
    d_ref[...] = _pack_pair(p[:, :256], p[:, 256:512])
    s_ref[...] = _pack_pair(p[:, 512:768], p[:, 768:])
    wrp_ref[...] = _pack_pair(wr_ref[0:1, :], wr_ref[1:2, :])


def _proj2_kernel(x_ref, agg_ref, w_ref, b_ref, wr_ref, h_ref, d_ref,
                  s_ref, wrp_ref):
    aggw = agg_ref[...]
    agg = jnp.concatenate([_unpack_lo(aggw), _unpack_hi(aggw)], axis=1)
    h = x_ref[...] + agg
    h = jnp.where(h > 0, h, jnp.exp(h) - 1.0)
    h_ref[...] = h
    p = jnp.dot(h, w_ref[...], preferred_element_type=jnp.float32)
    p = p + b_ref[...]
    d_ref[...] = _pack_pair(p[:, :256], p[:, 256:512])
    s_ref[...] = _pack_pair(p[:, 512:768], p[:, 768:])
    wrp_ref[...] = _pack_pair(wr_ref[0:1, :], wr_ref[1:2, :])


def _msg_kernel(z_ref, m_ref):
    zw = z_ref[...]
    zf = _unpack_lo(zw)
    zs = _unpack_hi(zw)
    sig = 1.0 / (1.0 + jnp.exp(-zf))
    t = jnp.exp(-jnp.abs(zs))
    sp = jnp.maximum(zs, 0.0) + jnp.log(1.0 + t)
    m = sig * sp
    m_ref[...] = _pack_pair(m[:, :128], m[:, 128:])


def _head_kernel(h_ref, agg_ref, w1_ref, b1_ref, w2_ref, b2_ref, o_ref):
    aggw = agg_ref[...]
    agg = jnp.concatenate([_unpack_lo(aggw), _unpack_hi(aggw)], axis=1)
    h2 = h_ref[...] + agg
    h2 = jnp.where(h2 > 0, h2, jnp.exp(h2) - 1.0)
    t = jnp.dot(h2, w1_ref[...], preferred_element_type=jnp.float32)
    t = t + b1_ref[...]
    t = jnp.where(t > 0, t, jnp.exp(t) - 1.0)
    o = jnp.dot(t, w2_ref[...], preferred_element_type=jnp.float32)
    o = o + b2_ref[...]
    lane = lax.broadcasted_iota(jnp.int32, o.shape, 1)
    valid = lane < 10
    neg = jnp.float32(-1e30)
    m = jnp.max(jnp.where(valid, o, neg), axis=1, keepdims=True)
    s = jnp.sum(jnp.where(valid, jnp.exp(o - m), 0.0), axis=1, keepdims=True)
    res = o - m - jnp.log(s)
    o_ref[...] = res[:, :10]


def _proj1(x, w, b, wr):
    return pl.pallas_call(
        _proj1_kernel,
        grid=(NPAD // 256,),
        in_specs=[pl.BlockSpec((256, 256), lambda i: (i, 0)),
                  pl.BlockSpec((256, 1024), lambda i: (0, 0)),
                  pl.BlockSpec((1, 1024), lambda i: (0, 0)),
                  pl.BlockSpec((2, 256), lambda i: (0, 0))],
        out_specs=[pl.BlockSpec((256, 256), lambda i: (i, 0)),
                   pl.BlockSpec((256, 256), lambda i: (i, 0)),
                   pl.BlockSpec((1, 256), lambda i: (0, 0))],
        out_shape=[jax.ShapeDtypeStruct((NPAD, 256), jnp.int32),
                   jax.ShapeDtypeStruct((NPAD, 256), jnp.int32),
                   jax.ShapeDtypeStruct((1, 256), jnp.int32)],
    )(x, w, b, wr)


def _proj2(x, agg, w, b, wr):
    return pl.pallas_call(
        _proj2_kernel,
        grid=(NPAD // 256,),
        in_specs=[pl.BlockSpec((256, 256), lambda i: (i, 0)),
                  pl.BlockSpec((256, 128), lambda i: (i, 0)),
                  pl.BlockSpec((256, 1024), lambda i: (0, 0)),
                  pl.BlockSpec((1, 1024), lambda i: (0, 0)),
                  pl.BlockSpec((2, 256), lambda i: (0, 0))],
        out_specs=[pl.BlockSpec((256, 256), lambda i: (i, 0)),
                   pl.BlockSpec((256, 256), lambda i: (i, 0)),
                   pl.BlockSpec((256, 256), lambda i: (i, 0)),
                   pl.BlockSpec((1, 256), lambda i: (0, 0))],
        out_shape=[jax.ShapeDtypeStruct((N, 256), jnp.float32),
                   jax.ShapeDtypeStruct((NPAD, 256), jnp.int32),
                   jax.ShapeDtypeStruct((NPAD, 256), jnp.int32),
                   jax.ShapeDtypeStruct((1, 256), jnp.int32)],
    )(x, agg, w, b, wr)


def _msg(z):
    return pl.pallas_call(
        _msg_kernel,
        grid=(EPAD // 512,),
        in_specs=[pl.BlockSpec((512, 256), lambda i: (i, 0))],
        out_specs=pl.BlockSpec((512, 128), lambda i: (i, 0)),
        out_shape=jax.ShapeDtypeStruct((EPAD, 128), jnp.int32),
    )(z)


def _head(h, agg, w1, b1, w2, b2):
    return pl.pallas_call(
        _head_kernel,
        grid=(NPAD // 256,),
        in_specs=[pl.BlockSpec((256, 256), lambda i: (i, 0)),
                  pl.BlockSpec((256, 128), lambda i: (i, 0)),
                  pl.BlockSpec((256, 128), lambda i: (0, 0)),
                  pl.BlockSpec((1, 128), lambda i: (0, 0)),
                  pl.BlockSpec((128, 128), lambda i: (0, 0)),
                  pl.BlockSpec((1, 128), lambda i: (0, 0))],
        out_specs=pl.BlockSpec((256, 10), lambda i: (i, 0)),
        out_shape=jax.ShapeDtypeStruct((N, 10), jnp.float32),
    )(h, agg, w1, b1, w2, b2)


def _expand_body(d_hbm, s_hbm, dst_hbm, src_hbm, ea_hbm, wrow_hbm, z_hbm,
                 idd, ids, eas, dbuf, sbuf, zbuf, wrowv, sem0, sem1,
                 zsem0, zsem1):
    wid = lax.axis_index("s") * 2 + lax.axis_index("c")
    e0 = wid * NEPW

    pltpu.sync_copy(dst_hbm.at[pl.ds(e0, NEPW)], idd)
    pltpu.sync_copy(src_hbm.at[pl.ds(e0, NEPW)], ids)
    pltpu.sync_copy(ea_hbm.at[pl.ds(e0, NEPW)], eas)
    pltpu.sync_copy(wrow_hbm, wrowv)

    sems = (sem0, sem1)
    zsems = (zsem0, zsem1)

    def issue(c, slot):
        dv = idd[pl.ds(c * 16, 16)]
        sv = ids[pl.ds(c * 16, 16)]
        pltpu.async_copy(d_hbm.at[dv], dbuf.at[slot], sems[slot])
        pltpu.async_copy(s_hbm.at[sv], sbuf.at[slot], sems[slot])

    def wait(slot):
        pltpu.make_async_copy(d_hbm.at[pl.ds(0, 16)], dbuf.at[slot],
                              sems[slot]).wait()
        pltpu.make_async_copy(s_hbm.at[pl.ds(0, 16)], sbuf.at[slot],
                              sems[slot]).wait()

    def zdrain(slot):
        pltpu.make_async_copy(zbuf.at[slot], z_hbm.at[pl.ds(0, 16)],
                              zsems[slot]).wait()

    def compute(c, slot):
        @pl.when(c >= 2)
        def _():
            zdrain(slot)

        def edge_j(j, carry):
            eab = plsc.load_gather(eas, [jnp.full((16,), 0, jnp.int32)
                                         + (c * 16 + j)])
            eab2 = plsc.pack(eab, eab, format=plsc.PackFormat.INTERLEAVED)
            for v in range(16):
                sl = pl.ds(v * 16, 16)
                wsl = wrowv[0, pl.ds(v * 16, 16)]
                db = plsc.bitcast(dbuf[slot, j, sl], jnp.bfloat16)
                sb = plsc.bitcast(sbuf[slot, j, sl], jnp.bfloat16)
                wb = plsc.bitcast(wsl, jnp.bfloat16)
                z = db + sb + eab2 * wb
                zbuf[slot, j, sl] = plsc.bitcast(z, jnp.int32)
            return carry

        lax.fori_loop(0, 16, edge_j, 0)
        pltpu.async_copy(zbuf.at[slot], z_hbm.at[pl.ds(e0 + c * 16, 16)],
                         zsems[slot])

    issue(0, 0)

    def pair(p, carry):
        c0 = p * 2
        issue(c0 + 1, 1)
        wait(0)
        compute(c0, 0)

        @pl.when(c0 + 2 < NCHUNK)
        def _():
            issue(c0 + 2, 0)

        wait(1)
        compute(c0 + 1, 1)
        return carry

    lax.fori_loop(0, NCHUNK // 2, pair, 0)
    zdrain(0)
    zdrain(1)


_expand = functools.partial(
    pl.kernel,
    out_type=jax.ShapeDtypeStruct((EPAD, 256), jnp.int32),
    mesh=plsc.VectorSubcoreMesh(core_axis_name="c", subcore_axis_name="s"),
    compiler_params=pltpu.CompilerParams(needs_layout_passes=False),
    scratch_types=[
        pltpu.VMEM((NEPW,), jnp.int32),
        pltpu.VMEM((NEPW,), jnp.int32),
        pltpu.VMEM((NEPW,), jnp.float32),
        pltpu.VMEM((2, 16, 256), jnp.int32),
        pltpu.VMEM((2, 16, 256), jnp.int32),
        pltpu.VMEM((2, 16, 256), jnp.int32),
        pltpu.VMEM((1, 256), jnp.int32),
        pltpu.SemaphoreType.DMA,
        pltpu.SemaphoreType.DMA,
        pltpu.SemaphoreType.DMA,
        pltpu.SemaphoreType.DMA,
    ],
)(_expand_body)


def _segmax_body(msg_hbm, dst_hbm, out_hbm,
                 staging, blk, pend, pendd, mbuf, sem):
    wid = lax.axis_index("s") * 2 + lax.axis_index("c")
    base = wid * NPW

    zeros16 = jnp.zeros((16,), jnp.int32)

    def zrow(i, carry):
        for v in range(8):
            staging[i, pl.ds(v * 16, 16)] = zeros16
        return carry

    lax.fori_loop(0, NPW + 1, zrow, 0)

    sent = jnp.full((16,), E, jnp.int32)
    sentd = jnp.full((16,), N, jnp.int32)
    for q in range(3):
        pend[pl.ds(q * 16, 16)] = sent
        pendd[pl.ds(q * 16, 16)] = sentd

    def rmw_group(g, goff):
        def edge_j(j, carry):
            idxj = jnp.full((16,), goff, jnp.int32) + j
            dv = plsc.load_gather(pendd, [idxj])
            d0 = lax.squeeze(lax.slice(dv, (0,), (1,)), (0,))
            row = jnp.minimum(d0 - base, NPW)
            for v in range(8):
                sl = pl.ds(v * 16, 16)
                sb = plsc.bitcast(staging[row, sl], jnp.bfloat16)
                mb = plsc.bitcast(mbuf[g, j, sl], jnp.bfloat16)
                staging[row, sl] = plsc.bitcast(jnp.maximum(sb, mb),
                                                jnp.int32)
            return carry

        lax.fori_loop(0, 16, edge_j, 0)

    def process_two():
        e0 = pend[pl.ds(0, 16)]
        e1 = pend[pl.ds(16, 16)]
        c1 = pltpu.async_copy(msg_hbm.at[e0], mbuf.at[0], sem)
        c3 = pltpu.async_copy(msg_hbm.at[e1], mbuf.at[1], sem)
        c1.wait()
        rmw_group(0, 0)
        c3.wait()
        rmw_group(1, 16)

    def process_one(goff, g):
        eid = pend[pl.ds(goff, 16)]
        cp2 = pltpu.async_copy(msg_hbm.at[eid], mbuf.at[g], sem)
        cp2.wait()
        rmw_group(g, goff)

    def block_b(b, cnt):
        pltpu.sync_copy(dst_hbm.at[pl.ds(b * BLK, BLK)], blk)

        def chunk(c, cnt):
            dch = blk[pl.ds(c * 16, 16)]
            mask = (dch >= base) & (dch < base + NPW)
            cs = jnp.cumsum(jnp.where(mask, 1, 0))
            npc = lax.squeeze(lax.slice(cs, (15,), (16,)), (0,))
            pos = cnt + cs - 1
            eidv = (b * BLK + c * 16) + lax.iota(jnp.int32, 16)
            plsc.store_scatter(pend, [pos], eidv, mask=mask)
            plsc.store_scatter(pendd, [pos], dch, mask=mask)
            cnt = cnt + npc
            pl.when(cnt >= 32)(process_two)

            def shift():
                pend[pl.ds(0, 16)] = pend[pl.ds(32, 16)]
                pendd[pl.ds(0, 16)] = pendd[pl.ds(32, 16)]

            pl.when(cnt >= 32)(shift)
            return jnp.where(cnt >= 32, cnt - 32, cnt)

        return lax.fori_loop(0, NCH, chunk, cnt)

    cnt = lax.fori_loop(0, NBLK, block_b, jnp.int32(0))
    pl.when(cnt > 0)(lambda: process_one(0, 0))
    pl.when(cnt > 16)(lambda: process_one(16, 1))

    @pl.when(wid < NW - 1)
    def _():
        pltpu.sync_copy(staging.at[pl.ds(0, NPW)], out_hbm.at[pl.ds(base, NPW)])

    @pl.when(wid == NW - 1)
    def _():
        pltpu.sync_copy(staging.at[pl.ds(0, N - (NW - 1) * NPW)],
                        out_hbm.at[pl.ds((NW - 1) * NPW, N - (NW - 1) * NPW)])


_segmax = functools.partial(
    pl.kernel,
    out_type=jax.ShapeDtypeStruct((N, 128), jnp.int32),
    mesh=plsc.VectorSubcoreMesh(core_axis_name="c", subcore_axis_name="s"),
    compiler_params=pltpu.CompilerParams(needs_layout_passes=False),
    scratch_types=[
        pltpu.VMEM((NPW + 1, 128), jnp.int32),
        pltpu.VMEM((BLK,), jnp.int32),
        pltpu.VMEM((48,), jnp.int32),
        pltpu.VMEM((48,), jnp.int32),
        pltpu.VMEM((2, 16, 128), jnp.int32),
        pltpu.SemaphoreType.DMA,
    ],
)(_segmax_body)


def _layer_agg(d, s, dstp, srcp, eap, wrowp):
    z = _expand(d, s, dstp, srcp, eap, wrowp)
    m = _msg(z)
    return _segmax(m, dstp)


def kernel(x, edge_index, edge_attr, Wf1, bf1, Ws1, bs1, Wf2, bf2, Ws2, bs2,
           Wfc1, bfc1, Wfc2, bfc2):
    src = edge_index[0]
    dst = edge_index[1]
    npad = EPAD - E
    dstp = jnp.concatenate([dst, jnp.full((npad,), N, jnp.int32)])
    srcp = jnp.concatenate([src, jnp.zeros((npad,), jnp.int32)])
    eap = jnp.concatenate([edge_attr[:, 0], jnp.zeros((npad,), jnp.float32)])

    def wcat(Wf, bf, Ws, bs):
        w = jnp.concatenate([Wf[:F], Ws[:F], Wf[F:2 * F], Ws[F:2 * F]], axis=1)
        b = jnp.concatenate([bf, bs, jnp.zeros((512,), jnp.float32)])[None, :]
        wrow = jnp.stack([Wf[2 * F], Ws[2 * F]])
        return w, b, wrow

    w1, b1, wrow1 = wcat(Wf1, bf1, Ws1, bs1)
    w2, b2, wrow2 = wcat(Wf2, bf2, Ws2, bs2)

    d1, s1, wrp1 = _proj1(x, w1, b1, wrow1)
    agg1 = _layer_agg(d1, s1, dstp, srcp, eap, wrp1)
    h1, d2, s2, wrp2 = _proj2(x, agg1, w2, b2, wrow2)
    agg2 = _layer_agg(d2, s2, dstp, srcp, eap, wrp2)

    w2p = jnp.zeros((128, 128), jnp.float32).at[:, :10].set(Wfc2)
    b2p = jnp.zeros((1, 128), jnp.float32).at[0, :10].set(bfc2)
    return _head(h1, agg2, Wfc1, bfc1[None, :], w2p, b2p)
